# Initial kernel scaffold; baseline (speedup 1.0000x reference)
import jax
import jax.numpy as jnp
from jax.experimental import pallas as pl


def kernel(idx, logits_table):
    raise NotImplementedError("write your pallas kernel here")


# speedup vs baseline: 1.0146x; 1.0146x over previous
import functools

import jax
import jax.numpy as jnp
from jax import lax
from jax.experimental import pallas as pl
from jax.experimental.pallas import tpu as pltpu
from jax.experimental.pallas import tpu_sc as plsc

VOCAB = 1000
ROW = 1000

NUM_CORES = 2
NUM_SUBCORES = 16
NW = NUM_CORES * NUM_SUBCORES

B_TOTAL = 1024 * 50
B_PER_W = B_TOTAL // NW
CHUNK = 64
N_CHUNKS = B_PER_W // CHUNK

_mesh = plsc.VectorSubcoreMesh(core_axis_name="c", subcore_axis_name="s")


@functools.partial(
    pl.kernel,
    mesh=_mesh,
    out_type=jax.ShapeDtypeStruct((B_TOTAL, ROW), jnp.float32),
    scratch_types=[
        pltpu.VMEM((B_PER_W,), jnp.int32),
        pltpu.VMEM((CHUNK, ROW), jnp.float32),
        pltpu.SemaphoreType.DMA,
    ],
    compiler_params=pltpu.CompilerParams(use_tc_tiling_on_sc=False),
)
def _gather_rows(table_hbm, idx_hbm, out_hbm, idx_v, rows_v, sem):
    wid = lax.axis_index("s") * NUM_CORES + lax.axis_index("c")
    base = wid * B_PER_W
    pltpu.sync_copy(idx_hbm.at[pl.ds(base, B_PER_W)], idx_v)

    def body(i, _):
        off = i * CHUNK
        pltpu.async_copy(
            table_hbm.at[idx_v.at[pl.ds(off, CHUNK)]], rows_v, sem
        ).wait()
        pltpu.sync_copy(rows_v, out_hbm.at[pl.ds(base + off, CHUNK)])
        return ()

    lax.fori_loop(0, N_CHUNKS, body, ())


def kernel(idx, logits_table):
    flat_idx = idx.reshape(-1).astype(jnp.int32)
    out = _gather_rows(logits_table, flat_idx)
    return out.reshape(idx.shape[0], idx.shape[1], VOCAB)


# speedup vs baseline: 1.0347x; 1.0198x over previous
import functools

import jax
import jax.numpy as jnp
from jax import lax
from jax.experimental import pallas as pl
from jax.experimental.pallas import tpu as pltpu
from jax.experimental.pallas import tpu_sc as plsc

VOCAB = 1000
ROW = 1000

NUM_CORES = 2
NUM_SUBCORES = 16
NW = NUM_CORES * NUM_SUBCORES

B_TOTAL = 1024 * 50
B_PER_W = B_TOTAL // NW
CHUNK = 40
N_BUF = 2
N_CHUNKS = B_PER_W // CHUNK
N_OUTER = N_CHUNKS // N_BUF

_mesh = plsc.VectorSubcoreMesh(core_axis_name="c", subcore_axis_name="s")


@functools.partial(
    pl.kernel,
    mesh=_mesh,
    out_type=jax.ShapeDtypeStruct((B_TOTAL, ROW), jnp.float32),
    scratch_types=[
        pltpu.VMEM((B_PER_W,), jnp.int32),
        pltpu.VMEM((N_BUF, CHUNK, ROW), jnp.float32),
        pltpu.SemaphoreType.DMA((N_BUF,)),
    ],
    compiler_params=pltpu.CompilerParams(use_tc_tiling_on_sc=False),
)
def _gather_rows(table_hbm, idx_hbm, out_hbm, idx_v, rows_v, gsem):
    wid = lax.axis_index("s") * NUM_CORES + lax.axis_index("c")
    base = wid * B_PER_W
    pltpu.sync_copy(idx_hbm.at[pl.ds(base, B_PER_W)], idx_v)

    def gather_desc(i, b):
        return pltpu.make_async_copy(
            table_hbm.at[idx_v.at[pl.ds(i * CHUNK, CHUNK)]],
            rows_v.at[b],
            gsem.at[b],
        )

    def writeback_sync(i, b):
        pltpu.sync_copy(rows_v.at[b], out_hbm.at[pl.ds(base + i * CHUNK, CHUNK)])

    for b in range(N_BUF):
        gather_desc(b, b).start()

    def outer(g, _):
        for b in range(N_BUF):
            i = g * N_BUF + b
            gather_desc(i, b).wait()
            writeback_sync(i, b)
            gather_desc(i + N_BUF, b).start()
        return ()

    lax.fori_loop(0, N_OUTER - 1, outer, ())

    last = (N_OUTER - 1) * N_BUF
    for b in range(N_BUF):
        gather_desc(last + b, b).wait()
        writeback_sync(last + b, b)


def kernel(idx, logits_table):
    flat_idx = idx.reshape(-1).astype(jnp.int32)
    out = _gather_rows(logits_table, flat_idx)
    return out.reshape(idx.shape[0], idx.shape[1], VOCAB)
